# baseline (speedup 1.0000x reference)
import functools

import jax
import jax.numpy as jnp
from jax import lax
from jax.experimental import pallas as pl
from jax.experimental.pallas import tpu as pltpu
from jax.experimental.pallas import tpu_sc as plsc

N = 10000
E = 320000
D = 128

NC = 2
NS = 16
NW = NC * NS
EPW = E // NW
E2 = E // 2
EPW2 = E2 // NW
GCH = 40
GNCHUNK = EPW2 // GCH
CH = 40
NCHUNK = EPW // CH
NRING = 5
NP = 10240
RPT = NP // NS


@functools.cache
def _sc_kernels():
    mesh = plsc.VectorSubcoreMesh(core_axis_name="c", subcore_axis_name="s")

    @functools.partial(
        pl.kernel,
        mesh=mesh,
        out_type=jax.ShapeDtypeStruct((E2, D), jnp.float32),
        scratch_types=[pltpu.VMEM((GCH,), jnp.int32) for _ in range(2 * NRING)]
        + [pltpu.VMEM((GCH, D), jnp.float32) for _ in range(2 * NRING)]
        + [pltpu.SemaphoreType.DMA for _ in range(5 * NRING)],
    )
    def sc_gather(xs_hbm, xd_hbm, row_hbm, col_hbm, sum_out, *bufs_and_sems):
        ir = bufs_and_sems[:NRING]
        ic = bufs_and_sems[NRING:2 * NRING]
        bufs = bufs_and_sems[2 * NRING:4 * NRING]
        sems = bufs_and_sems[4 * NRING:]
        bs = bufs[:NRING]
        bd = bufs[NRING:]
        gs = sems[:NRING]
        gd = sems[NRING:2 * NRING]
        ss = sems[2 * NRING:3 * NRING]
        si = sems[3 * NRING:4 * NRING]
        sj = sems[4 * NRING:]

        wid = lax.axis_index("s") * NC + lax.axis_index("c")
        base = wid * EPW2

        def load_idx(j, p):
            off = base + j * GCH
            pltpu.async_copy(row_hbm.at[pl.ds(off, GCH)], ir[p], si[p])
            pltpu.async_copy(col_hbm.at[pl.ds(off, GCH)], ic[p], sj[p])

        def wait_idx(p):
            pltpu.make_async_copy(row_hbm.at[pl.ds(base, GCH)], ir[p], si[p]).wait()
            pltpu.make_async_copy(col_hbm.at[pl.ds(base, GCH)], ic[p], sj[p]).wait()

        def gather(p):
            pltpu.async_copy(xs_hbm.at[ir[p]], bs[p], gs[p])
            pltpu.async_copy(xd_hbm.at[ic[p]], bd[p], gd[p])

        for p in range(NRING):
            load_idx(p, p)
        for p in range(NRING):
            wait_idx(p)
            gather(p)

        def body(t, carry):
            for p in range(NRING):
                j = t * NRING + p
                off = base + j * GCH
                pltpu.make_async_copy(xs_hbm.at[ir[p]], bs[p], gs[p]).wait()
                pltpu.make_async_copy(xd_hbm.at[ic[p]], bd[p], gd[p]).wait()

                def add_row(r, carry2, _bs=bs[p], _bd=bd[p]):
                    for c in range(D // 16):
                        sl = pl.ds(c * 16, 16)
                        _bs[r, sl] = _bs[r, sl] + _bd[r, sl]
                    return carry2

                lax.fori_loop(0, GCH, add_row, 0)
                pltpu.async_copy(bs[p], sum_out.at[pl.ds(off, GCH)], ss[p])

                @pl.when(j + NRING < GNCHUNK)
                def _():
                    load_idx(j + NRING, p)
                    pltpu.make_async_copy(bs[p], sum_out.at[pl.ds(base, GCH)], ss[p]).wait()
                    wait_idx(p)
                    gather(p)

            return carry

        lax.fori_loop(0, GNCHUNK // NRING, body, 0)
        for p in range(NRING):
            pltpu.make_async_copy(bs[p], sum_out.at[pl.ds(base, GCH)], ss[p]).wait()

    @functools.partial(
        pl.kernel,
        mesh=mesh,
        out_type=jax.ShapeDtypeStruct((NC * NP, D), jnp.float32),
        scratch_types=[pltpu.VMEM_SHARED((NP, D), jnp.float32)]
        + [pltpu.VMEM((CH,), jnp.int32) for _ in range(NRING)]
        + [pltpu.VMEM((CH, D), jnp.float32) for _ in range(NRING)]
        + [pltpu.SemaphoreType.DMA for _ in range(3 * NRING)],
    )
    def sc_scatter(e_hbm, col_hbm, zeros_hbm, part_out, acc, *bufs_and_sems):
        ib = bufs_and_sems[:NRING]
        rb = bufs_and_sems[NRING:2 * NRING]
        sems = bufs_and_sems[2 * NRING:]
        si = sems[:NRING]
        sr = sems[NRING:2 * NRING]
        sa = sems[2 * NRING:]

        cid = lax.axis_index("c")
        sid = lax.axis_index("s")
        base = (sid * NC + cid) * EPW
        r0 = sid * RPT

        pltpu.sync_copy(zeros_hbm.at[pl.ds(r0, RPT)], acc.at[pl.ds(r0, RPT)])
        plsc.subcore_barrier()

        def load(j, p):
            off = base + j * CH
            pltpu.async_copy(col_hbm.at[pl.ds(off, CH)], ib[p], si[p])
            pltpu.async_copy(e_hbm.at[pl.ds(off, CH)], rb[p], sr[p])

        for p in range(NRING):
            load(p, p)

        def body(t, carry):
            for p in range(NRING):
                j = t * NRING + p
                pltpu.make_async_copy(col_hbm.at[pl.ds(base, CH)], ib[p], si[p]).wait()
                pltpu.make_async_copy(e_hbm.at[pl.ds(base, CH)], rb[p], sr[p]).wait()
                pltpu.async_copy(rb[p], acc.at[ib[p]], sa[p], add=True)

                @pl.when(j + NRING < NCHUNK)
                def _():
                    pltpu.make_async_copy(rb[p], acc.at[ib[p]], sa[p]).wait()
                    load(j + NRING, p)

            return carry

        lax.fori_loop(0, NCHUNK // NRING, body, 0)
        for p in range(NRING):
            pltpu.make_async_copy(rb[p], acc.at[ib[p]], sa[p]).wait()
        plsc.subcore_barrier()
        pltpu.sync_copy(acc.at[pl.ds(r0, RPT)],
                        part_out.at[pl.ds(cid * NP + r0, RPT)])

    return sc_gather, sc_scatter


def _ln(h, g, bb):
    mu = jnp.mean(h, axis=-1, keepdims=True)
    var = jnp.mean((h - mu) ** 2, axis=-1, keepdims=True)
    return (h - mu) * lax.rsqrt(var + 1e-5) * g + bb


def _prep_body(x_ref, ws_ref, wd_ref, xs_ref, xd_ref):
    xv = x_ref[...]
    xs_ref[...] = jnp.dot(xv, ws_ref[...], preferred_element_type=jnp.float32)
    xd_ref[...] = jnp.dot(xv, wd_ref[...], preferred_element_type=jnp.float32)


def _edge_body(sum_ref, e_ref, w0_ref, b0_ref, w1_ref, b1_ref,
               w2_ref, b2_ref, g_ref, bb_ref, out_ref):
    h = (sum_ref[...] + b0_ref[...]
         + jnp.dot(e_ref[...], w0_ref[...], preferred_element_type=jnp.float32))
    h = jnp.maximum(h, 0.0)
    h = jnp.maximum(
        jnp.dot(h, w1_ref[...], preferred_element_type=jnp.float32) + b1_ref[...], 0.0)
    h = jnp.dot(h, w2_ref[...], preferred_element_type=jnp.float32) + b2_ref[...]
    out_ref[...] = _ln(h, g_ref[...], bb_ref[...])


def _node_body(x_ref, p0_ref, p1_ref, wx_ref, wa_ref, b0_ref, w1_ref, b1_ref,
               w2_ref, b2_ref, g_ref, bb_ref, out_ref):
    agg = p0_ref[...] + p1_ref[...]
    h = (jnp.dot(x_ref[...], wx_ref[...], preferred_element_type=jnp.float32)
         + jnp.dot(agg, wa_ref[...], preferred_element_type=jnp.float32)
         + b0_ref[...])
    h = jnp.maximum(h, 0.0)
    h = jnp.maximum(
        jnp.dot(h, w1_ref[...], preferred_element_type=jnp.float32) + b1_ref[...], 0.0)
    h = jnp.dot(h, w2_ref[...], preferred_element_type=jnp.float32) + b2_ref[...]
    out_ref[...] = _ln(h, g_ref[...], bb_ref[...])


BN = 1000
BE = 2000

_w = pl.BlockSpec((D, D), lambda i: (0, 0))
_v = pl.BlockSpec((1, D), lambda i: (0, 0))
_nblk = pl.BlockSpec((BN, D), lambda i: (i, 0))
_eblk = pl.BlockSpec((BE, D), lambda i: (i, 0))


def _prep(x, ws, wd):
    return pl.pallas_call(
        _prep_body,
        grid=(N // BN,),
        in_specs=[_nblk, _w, _w],
        out_specs=(_nblk, _nblk),
        out_shape=(jax.ShapeDtypeStruct((N, D), jnp.float32),
                   jax.ShapeDtypeStruct((N, D), jnp.float32)),
    )(x, ws, wd)


NB2 = E2 // BE


def _edge_mlp_half(half, sumb, e, w0, b0, w1, b1, w2, b2, g, bb):
    hblk = pl.BlockSpec((BE, D), lambda i: (i, 0))
    fblk = pl.BlockSpec((BE, D), lambda i, h=half: (i + h * NB2, 0))
    return pl.pallas_call(
        _edge_body,
        grid=(NB2,),
        in_specs=[hblk, fblk, _w, _v, _w, _v, _w, _v, _v, _v],
        out_specs=fblk,
        out_shape=jax.ShapeDtypeStruct((E, D), jnp.float32),
        input_output_aliases={1: 0},
    )(sumb, e, w0, b0, w1, b1, w2, b2, g, bb)


def _node_mlp(x, p0, p1, wx, wa, b0, w1, b1, w2, b2, g, bb):
    return pl.pallas_call(
        _node_body,
        grid=(N // BN,),
        in_specs=[_nblk, _nblk, _nblk, _w, _w, _v, _w, _v, _w, _v, _v, _v],
        out_specs=_nblk,
        out_shape=jax.ShapeDtypeStruct((N, D), jnp.float32),
    )(x, p0, p1, wx, wa, b0, w1, b1, w2, b2, g, bb)


def kernel(x, edge_attr, edge_index, eW0, eb0, eW1, eb1, eW2, eb2, eg, ebb,
           nW0, nb0, nW1, nb1, nW2, nb2, ng, nbb):
    row = edge_index[0]
    col = edge_index[1]
    zeros = jnp.zeros((NP, D), jnp.float32)
    r1 = lambda a: a.reshape(1, D)
    _sc_gather, _sc_scatter = _sc_kernels()

    for i in range(eW0.shape[0]):
        ws, wd, we = eW0[i, :D], eW0[i, D:2 * D], eW0[i, 2 * D:]
        xs, xd = _prep(x, ws, wd)
        sumA = _sc_gather(xs, xd, row[:E2], col[:E2])
        sumB = _sc_gather(xs, xd, row[E2:], col[E2:])
        eargs = (we, r1(eb0[i]), eW1[i], r1(eb1[i]), eW2[i], r1(eb2[i]),
                 r1(eg[i]), r1(ebb[i]))
        edge_attr = _edge_mlp_half(0, sumA, edge_attr, *eargs)
        edge_attr = _edge_mlp_half(1, sumB, edge_attr, *eargs)
        parts = _sc_scatter(edge_attr, col, zeros)
        x = _node_mlp(x, parts[:N], parts[NP:NP + N], nW0[i, :D], nW0[i, D:],
                      r1(nb0[i]), nW1[i], r1(nb1[i]), nW2[i], r1(nb2[i]),
                      r1(ng[i]), r1(nbb[i]))
    return x, edge_attr


# speedup vs baseline: 1.0474x; 1.0474x over previous
import functools

import jax
import jax.numpy as jnp
from jax import lax
from jax.experimental import pallas as pl
from jax.experimental.pallas import tpu as pltpu
from jax.experimental.pallas import tpu_sc as plsc

N = 10000
E = 320000
D = 128

NC = 2
NS = 16
NW = NC * NS
EPW = E // NW
GCH = 80
GNCHUNK = EPW // GCH
CH = 40
NCHUNK = EPW // CH
NRING = 5
NP = 10240
RPT = NP // NS


@functools.cache
def _sc_kernels():
    mesh = plsc.VectorSubcoreMesh(core_axis_name="c", subcore_axis_name="s")

    @functools.partial(
        pl.kernel,
        mesh=mesh,
        out_type=jax.ShapeDtypeStruct((E, D), jnp.float32),
        scratch_types=[pltpu.VMEM((GCH,), jnp.int32) for _ in range(2 * NRING)]
        + [pltpu.VMEM((GCH, D), jnp.float32) for _ in range(2 * NRING)]
        + [pltpu.SemaphoreType.DMA for _ in range(5 * NRING)],
    )
    def sc_gather(xs_hbm, xd_hbm, row_hbm, col_hbm, sum_out, *bufs_and_sems):
        ir = bufs_and_sems[:NRING]
        ic = bufs_and_sems[NRING:2 * NRING]
        bufs = bufs_and_sems[2 * NRING:4 * NRING]
        sems = bufs_and_sems[4 * NRING:]
        bs = bufs[:NRING]
        bd = bufs[NRING:]
        gs = sems[:NRING]
        gd = sems[NRING:2 * NRING]
        ss = sems[2 * NRING:3 * NRING]
        si = sems[3 * NRING:4 * NRING]
        sj = sems[4 * NRING:]

        wid = lax.axis_index("s") * NC + lax.axis_index("c")
        base = wid * EPW

        def load_idx(j, p):
            off = base + j * GCH
            pltpu.async_copy(row_hbm.at[pl.ds(off, GCH)], ir[p], si[p])
            pltpu.async_copy(col_hbm.at[pl.ds(off, GCH)], ic[p], sj[p])

        def wait_idx(p):
            pltpu.make_async_copy(row_hbm.at[pl.ds(base, GCH)], ir[p], si[p]).wait()
            pltpu.make_async_copy(col_hbm.at[pl.ds(base, GCH)], ic[p], sj[p]).wait()

        def gather(p):
            pltpu.async_copy(xs_hbm.at[ir[p]], bs[p], gs[p])
            pltpu.async_copy(xd_hbm.at[ic[p]], bd[p], gd[p])

        for p in range(NRING):
            load_idx(p, p)
        for p in range(NRING):
            wait_idx(p)
            gather(p)

        def body(t, carry):
            for p in range(NRING):
                j = t * NRING + p
                off = base + j * GCH
                pltpu.make_async_copy(xs_hbm.at[ir[p]], bs[p], gs[p]).wait()
                pltpu.make_async_copy(xd_hbm.at[ic[p]], bd[p], gd[p]).wait()

                def add_row(r, carry2, _bs=bs[p], _bd=bd[p]):
                    for c in range(D // 16):
                        sl = pl.ds(c * 16, 16)
                        _bs[r, sl] = _bs[r, sl] + _bd[r, sl]
                    return carry2

                lax.fori_loop(0, GCH, add_row, 0)
                pltpu.async_copy(bs[p], sum_out.at[pl.ds(off, GCH)], ss[p])

                @pl.when(j + NRING < GNCHUNK)
                def _():
                    load_idx(j + NRING, p)
                    pltpu.make_async_copy(bs[p], sum_out.at[pl.ds(base, GCH)], ss[p]).wait()
                    wait_idx(p)
                    gather(p)

            return carry

        lax.fori_loop(0, GNCHUNK // NRING, body, 0)
        for p in range(NRING):
            pltpu.make_async_copy(bs[p], sum_out.at[pl.ds(base, GCH)], ss[p]).wait()

    @functools.partial(
        pl.kernel,
        mesh=mesh,
        out_type=jax.ShapeDtypeStruct((NC * NP, D), jnp.float32),
        scratch_types=[pltpu.VMEM_SHARED((NP, D), jnp.float32)]
        + [pltpu.VMEM((CH,), jnp.int32) for _ in range(NRING)]
        + [pltpu.VMEM((CH, D), jnp.float32) for _ in range(NRING)]
        + [pltpu.SemaphoreType.DMA for _ in range(3 * NRING)],
    )
    def sc_scatter(e_hbm, col_hbm, zeros_hbm, part_out, acc, *bufs_and_sems):
        ib = bufs_and_sems[:NRING]
        rb = bufs_and_sems[NRING:2 * NRING]
        sems = bufs_and_sems[2 * NRING:]
        si = sems[:NRING]
        sr = sems[NRING:2 * NRING]
        sa = sems[2 * NRING:]

        cid = lax.axis_index("c")
        sid = lax.axis_index("s")
        base = (sid * NC + cid) * EPW
        r0 = sid * RPT

        pltpu.sync_copy(zeros_hbm.at[pl.ds(r0, RPT)], acc.at[pl.ds(r0, RPT)])
        plsc.subcore_barrier()

        def load(j, p):
            off = base + j * CH
            pltpu.async_copy(col_hbm.at[pl.ds(off, CH)], ib[p], si[p])
            pltpu.async_copy(e_hbm.at[pl.ds(off, CH)], rb[p], sr[p])

        for p in range(NRING):
            load(p, p)

        def body(t, carry):
            for p in range(NRING):
                j = t * NRING + p
                pltpu.make_async_copy(col_hbm.at[pl.ds(base, CH)], ib[p], si[p]).wait()
                pltpu.make_async_copy(e_hbm.at[pl.ds(base, CH)], rb[p], sr[p]).wait()
                pltpu.async_copy(rb[p], acc.at[ib[p]], sa[p], add=True)

                @pl.when(j + NRING < NCHUNK)
                def _():
                    pltpu.make_async_copy(rb[p], acc.at[ib[p]], sa[p]).wait()
                    load(j + NRING, p)

            return carry

        lax.fori_loop(0, NCHUNK // NRING, body, 0)
        for p in range(NRING):
            pltpu.make_async_copy(rb[p], acc.at[ib[p]], sa[p]).wait()
        plsc.subcore_barrier()
        pltpu.sync_copy(acc.at[pl.ds(r0, RPT)],
                        part_out.at[pl.ds(cid * NP + r0, RPT)])

    return sc_gather, sc_scatter


def _ln(h, g, bb):
    mu = jnp.mean(h, axis=-1, keepdims=True)
    var = jnp.mean((h - mu) ** 2, axis=-1, keepdims=True)
    return (h - mu) * lax.rsqrt(var + 1e-5) * g + bb


def _prep_body(x_ref, ws_ref, wd_ref, xs_ref, xd_ref):
    xv = x_ref[...]
    xs_ref[...] = jnp.dot(xv, ws_ref[...], preferred_element_type=jnp.float32)
    xd_ref[...] = jnp.dot(xv, wd_ref[...], preferred_element_type=jnp.float32)


def _edge_body(sum_ref, e_ref, w0_ref, b0_ref, w1_ref, b1_ref,
               w2_ref, b2_ref, g_ref, bb_ref, out_ref):
    h = (sum_ref[...] + b0_ref[...]
         + jnp.dot(e_ref[...], w0_ref[...], preferred_element_type=jnp.float32))
    h = jnp.maximum(h, 0.0)
    h = jnp.maximum(
        jnp.dot(h, w1_ref[...], preferred_element_type=jnp.float32) + b1_ref[...], 0.0)
    h = jnp.dot(h, w2_ref[...], preferred_element_type=jnp.float32) + b2_ref[...]
    out_ref[...] = _ln(h, g_ref[...], bb_ref[...])


def _node_body(x_ref, p0_ref, p1_ref, wx_ref, wa_ref, b0_ref, w1_ref, b1_ref,
               w2_ref, b2_ref, g_ref, bb_ref, out_ref):
    agg = p0_ref[...] + p1_ref[...]
    h = (jnp.dot(x_ref[...], wx_ref[...], preferred_element_type=jnp.float32)
         + jnp.dot(agg, wa_ref[...], preferred_element_type=jnp.float32)
         + b0_ref[...])
    h = jnp.maximum(h, 0.0)
    h = jnp.maximum(
        jnp.dot(h, w1_ref[...], preferred_element_type=jnp.float32) + b1_ref[...], 0.0)
    h = jnp.dot(h, w2_ref[...], preferred_element_type=jnp.float32) + b2_ref[...]
    out_ref[...] = _ln(h, g_ref[...], bb_ref[...])


BN = 1000
BE = 2000

_w = pl.BlockSpec((D, D), lambda i: (0, 0))
_v = pl.BlockSpec((1, D), lambda i: (0, 0))
_nblk = pl.BlockSpec((BN, D), lambda i: (i, 0))
_eblk = pl.BlockSpec((BE, D), lambda i: (i, 0))


def _prep(x, ws, wd):
    return pl.pallas_call(
        _prep_body,
        grid=(N // BN,),
        in_specs=[_nblk, _w, _w],
        out_specs=(_nblk, _nblk),
        out_shape=(jax.ShapeDtypeStruct((N, D), jnp.float32),
                   jax.ShapeDtypeStruct((N, D), jnp.float32)),
    )(x, ws, wd)


def _edge_mlp(sumb, e, w0, b0, w1, b1, w2, b2, g, bb):
    return pl.pallas_call(
        _edge_body,
        grid=(E // BE,),
        in_specs=[_eblk, _eblk, _w, _v, _w, _v, _w, _v, _v, _v],
        out_specs=_eblk,
        out_shape=jax.ShapeDtypeStruct((E, D), jnp.float32),
    )(sumb, e, w0, b0, w1, b1, w2, b2, g, bb)


def _node_mlp(x, p0, p1, wx, wa, b0, w1, b1, w2, b2, g, bb):
    return pl.pallas_call(
        _node_body,
        grid=(N // BN,),
        in_specs=[_nblk, _nblk, _nblk, _w, _w, _v, _w, _v, _w, _v, _v, _v],
        out_specs=_nblk,
        out_shape=jax.ShapeDtypeStruct((N, D), jnp.float32),
    )(x, p0, p1, wx, wa, b0, w1, b1, w2, b2, g, bb)


def kernel(x, edge_attr, edge_index, eW0, eb0, eW1, eb1, eW2, eb2, eg, ebb,
           nW0, nb0, nW1, nb1, nW2, nb2, ng, nbb):
    row = edge_index[0]
    col = edge_index[1]
    zeros = jnp.zeros((NP, D), jnp.float32)
    r1 = lambda a: a.reshape(1, D)
    _sc_gather, _sc_scatter = _sc_kernels()

    for i in range(eW0.shape[0]):
        ws, wd, we = eW0[i, :D], eW0[i, D:2 * D], eW0[i, 2 * D:]
        xs, xd = _prep(x, ws, wd)
        sumb = _sc_gather(xs, xd, row, col)
        edge_attr = _edge_mlp(sumb, edge_attr, we, r1(eb0[i]),
                              eW1[i], r1(eb1[i]), eW2[i], r1(eb2[i]),
                              r1(eg[i]), r1(ebb[i]))
        parts = _sc_scatter(edge_attr, col, zeros)
        x = _node_mlp(x, parts[:N], parts[NP:NP + N], nW0[i, :D], nW0[i, D:],
                      r1(nb0[i]), nW1[i], r1(nb1[i]), nW2[i], r1(nb2[i]),
                      r1(ng[i]), r1(nbb[i]))
    return x, edge_attr
